# Initial kernel scaffold; baseline (speedup 1.0000x reference)
import jax
import jax.numpy as jnp
from jax.experimental import pallas as pl


def kernel(x, hyperedge_matrix, point_hyperedge_index, centers, W1, b1, W2, b2, eps):
    raise NotImplementedError("write your pallas kernel here")


# speedup vs baseline: 8047.8911x; 8047.8911x over previous
import jax
import jax.numpy as jnp
from jax.experimental import pallas as pl
from jax.experimental.pallas import tpu as pltpu

B, C, H, W = 8, 768, 16, 16
N = H * W
HE, KN, KE = 64, 32, 3
COUT = 768


def _tc_body(hm_ref, phi_ref, x_ref, w1_ref, b1_ref, w2_ref, b2_ref, eps_ref,
             o_ref):
    f32 = jnp.float32
    xm = x_ref[0]

    iota_n = jax.lax.broadcasted_iota(jnp.int32, (HE, N), 1)
    a = jnp.zeros((HE, N), f32)
    for k in range(KN):
        row = hm_ref[0, k, :]
        a = a + jnp.where(row[:, None] == iota_n, 1.0 / KN, 0.0)

    he = jax.lax.dot_general(xm, a, (((1,), (1,)), ((), ())),
                             preferred_element_type=f32)
    h1 = jnp.maximum(
        jnp.dot(w1_ref[...], he, preferred_element_type=f32)
        + b1_ref[0][:, None], 0.0)

    iota_e = jax.lax.broadcasted_iota(jnp.int32, (HE, N), 0)
    p = jnp.zeros((HE, N), f32)
    for j in range(KE):
        row = phi_ref[0, j, :]
        p = p + jnp.where(row[None, :] == iota_e, 1.0 / KE, 0.0)

    nf = jnp.dot(h1, p, preferred_element_type=f32)
    y = (1.0 + eps_ref[0, 0]) * xm + nf
    out = jnp.maximum(
        jnp.dot(w2_ref[...], y, preferred_element_type=f32)
        + b2_ref[0][:, None], 0.0)
    o_ref[0] = out


def kernel(x, hyperedge_matrix, point_hyperedge_index, centers, W1, b1, W2,
           b2, eps):
    del centers
    xf = x.reshape(B, C, N)
    hm_t = jnp.transpose(hyperedge_matrix, (0, 2, 1))
    phi_t = jnp.transpose(point_hyperedge_index, (0, 2, 1))
    b1r = b1.reshape(1, C)
    b2r = b2.reshape(1, COUT)
    epsr = eps.reshape(1, 1)

    out = pl.pallas_call(
        _tc_body,
        grid=(B,),
        in_specs=[
            pl.BlockSpec((1, KN, HE), lambda b: (b, 0, 0)),
            pl.BlockSpec((1, KE, N), lambda b: (b, 0, 0)),
            pl.BlockSpec((1, C, N), lambda b: (b, 0, 0)),
            pl.BlockSpec((COUT, C), lambda b: (0, 0)),
            pl.BlockSpec((1, C), lambda b: (0, 0)),
            pl.BlockSpec((COUT, C), lambda b: (0, 0)),
            pl.BlockSpec((1, COUT), lambda b: (0, 0)),
            pl.BlockSpec((1, 1), lambda b: (0, 0), memory_space=pltpu.SMEM),
        ],
        out_specs=pl.BlockSpec((1, COUT, N), lambda b: (b, 0, 0)),
        out_shape=jax.ShapeDtypeStruct((B, COUT, N), jnp.float32),
    )(hm_t, phi_t, xf, W1, b1r, W2, b2r, epsr)
    return out.reshape(B, COUT, H, W)
